# baseline (speedup 1.0000x reference)
import functools

import jax
import jax.numpy as jnp
from jax import lax
from jax.experimental import pallas as pl
from jax.experimental.pallas import tpu as pltpu
from jax.experimental.pallas import tpu_sc as plsc

MAX_B = 8
MAX_S = 2048
N_HEADS = 16
HEAD_DIM = 128
Q_LEN = 16

ROWS = MAX_B * N_HEADS * MAX_S
SLABS = MAX_B * N_HEADS
VAL_ROWS = SLABS * Q_LEN

_info = plsc.get_sparse_core_info()
NC, NS, L = _info.num_cores, _info.num_subcores, _info.num_lanes
NW = NC * NS
ROWS_PER_W = ROWS // NW
SLABS_PER_W = SLABS // NW
CHUNK = 256
NCHUNK = ROWS_PER_W // CHUNK


def _copy_range(src, dst, base, buf0, buf1, sem_r, sem_w):
    bufs = (buf0, buf1)

    def read(c, buf):
        return pltpu.make_async_copy(src.at[pl.ds(base + c * CHUNK, CHUNK)], buf, sem_r)

    def write(c, buf):
        return pltpu.make_async_copy(buf, dst.at[pl.ds(base + c * CHUNK, CHUNK)], sem_w)

    read(0, buf0).start()
    read(1, buf1).start()

    def body(g):
        for b in range(2):
            c = g + b
            buf = bufs[b]
            read(c, buf).wait()
            write(c, buf).start()
        for b in range(2):
            c = g + b
            buf = bufs[b]
            write(c, buf).wait()

            @pl.when(c + 2 < NCHUNK)
            def _():
                read(c + 2, buf).start()

    pl.loop(0, NCHUNK, step=2)(body)


def _sc_body(kc, vc, pos, kv, vv, ko, vo,
             buf0, buf1, valbuf, idxbuf, posbuf, sem_r, sem_w, sem_s):
    wid = lax.axis_index("s") * NC + lax.axis_index("c")
    base = wid * ROWS_PER_W

    pltpu.sync_copy(pos, posbuf)
    p16 = posbuf[...]
    for s in range(SLABS_PER_W):
        slab = wid * SLABS_PER_W + s
        idxbuf[pl.ds(s * Q_LEN, Q_LEN)] = p16 + slab * MAX_S

    for src, valsrc, dst in ((kc, kv, ko), (vc, vv, vo)):
        _copy_range(src, dst, base, buf0, buf1, sem_r, sem_w)
        pltpu.async_copy(valsrc.at[pl.ds(wid * SLABS_PER_W * Q_LEN,
                                         SLABS_PER_W * Q_LEN)],
                         valbuf, sem_s).wait()
        pltpu.async_copy(valbuf, dst.at[idxbuf], sem_s).wait()


def kernel(k_cache, v_cache, input_pos, k_val, v_val):
    shape4 = (MAX_B, N_HEADS, MAX_S, HEAD_DIM)
    out_t = jax.ShapeDtypeStruct((ROWS, HEAD_DIM), jnp.float32)
    mesh = plsc.VectorSubcoreMesh(core_axis_name="c", subcore_axis_name="s")
    run = pl.kernel(
        _sc_body,
        out_type=[out_t, out_t],
        mesh=mesh,
        scratch_types=[
            pltpu.VMEM((CHUNK, HEAD_DIM), jnp.float32),
            pltpu.VMEM((CHUNK, HEAD_DIM), jnp.float32),
            pltpu.VMEM((SLABS_PER_W * Q_LEN, HEAD_DIM), jnp.float32),
            pltpu.VMEM((SLABS_PER_W * Q_LEN,), jnp.int32),
            pltpu.VMEM((Q_LEN,), jnp.int32),
            pltpu.SemaphoreType.DMA,
            pltpu.SemaphoreType.DMA,
            pltpu.SemaphoreType.DMA,
        ],
    )
    k_out, v_out = run(k_cache.reshape(ROWS, HEAD_DIM),
                       v_cache.reshape(ROWS, HEAD_DIM),
                       input_pos,
                       k_val.reshape(VAL_ROWS, HEAD_DIM),
                       v_val.reshape(VAL_ROWS, HEAD_DIM))
    return k_out.reshape(shape4), v_out.reshape(shape4)


# speedup vs baseline: 1.0061x; 1.0061x over previous
import functools

import jax
import jax.numpy as jnp
from jax import lax
from jax.experimental import pallas as pl
from jax.experimental.pallas import tpu as pltpu
from jax.experimental.pallas import tpu_sc as plsc

MAX_B = 8
MAX_S = 2048
N_HEADS = 16
HEAD_DIM = 128
Q_LEN = 16

ROWS = MAX_B * N_HEADS * MAX_S
SLABS = MAX_B * N_HEADS
VAL_ROWS = SLABS * Q_LEN

_info = plsc.get_sparse_core_info()
NC, NS, L = _info.num_cores, _info.num_subcores, _info.num_lanes
NW = NC * NS
ROWS_PER_W = ROWS // NW
SLABS_PER_W = SLABS // NW
CHUNK = 256
NCHUNK = ROWS_PER_W // CHUNK


def _copy_range(src, dst, base, bufs, sem_r, sem_w):
    nbuf = len(bufs)

    def read(c, buf):
        return pltpu.make_async_copy(src.at[pl.ds(base + c * CHUNK, CHUNK)], buf, sem_r)

    def write(c, buf):
        return pltpu.make_async_copy(buf, dst.at[pl.ds(base + c * CHUNK, CHUNK)], sem_w)

    for b in range(nbuf):
        read(b, bufs[b]).start()

    def body(g):
        for b in range(nbuf):
            c = g + b
            buf = bufs[b]
            read(c, buf).wait()
            write(c, buf).start()
        for b in range(nbuf):
            c = g + b
            buf = bufs[b]
            write(c, buf).wait()

            @pl.when(c + nbuf < NCHUNK)
            def _():
                read(c + nbuf, buf).start()

    main = (NCHUNK // nbuf) * nbuf
    pl.loop(0, main, step=nbuf)(body)
    for c in range(main, NCHUNK):
        buf = bufs[c % nbuf]
        read(c, buf).wait()
        write(c, buf).start()
    for c in range(main, NCHUNK):
        write(c, bufs[c % nbuf]).wait()


def _sc_body(kc, vc, pos, kv, vv, ko, vo,
             buf0, buf1, buf2, valbuf, idxbuf, posbuf, sem_r, sem_w, sem_s):
    wid = lax.axis_index("s") * NC + lax.axis_index("c")
    base = wid * ROWS_PER_W

    pltpu.sync_copy(pos, posbuf)
    p16 = posbuf[...]
    for s in range(SLABS_PER_W):
        slab = wid * SLABS_PER_W + s
        idxbuf[pl.ds(s * Q_LEN, Q_LEN)] = p16 + slab * MAX_S

    for src, valsrc, dst in ((kc, kv, ko), (vc, vv, vo)):
        _copy_range(src, dst, base, (buf0, buf1, buf2), sem_r, sem_w)
        pltpu.async_copy(valsrc.at[pl.ds(wid * SLABS_PER_W * Q_LEN,
                                         SLABS_PER_W * Q_LEN)],
                         valbuf, sem_s).wait()
        pltpu.async_copy(valbuf, dst.at[idxbuf], sem_s).wait()


def kernel(k_cache, v_cache, input_pos, k_val, v_val):
    shape4 = (MAX_B, N_HEADS, MAX_S, HEAD_DIM)
    out_t = jax.ShapeDtypeStruct((ROWS, HEAD_DIM), jnp.float32)
    mesh = plsc.VectorSubcoreMesh(core_axis_name="c", subcore_axis_name="s")
    run = pl.kernel(
        _sc_body,
        out_type=[out_t, out_t],
        mesh=mesh,
        scratch_types=[
            pltpu.VMEM((CHUNK, HEAD_DIM), jnp.float32),
            pltpu.VMEM((CHUNK, HEAD_DIM), jnp.float32),
            pltpu.VMEM((CHUNK, HEAD_DIM), jnp.float32),
            pltpu.VMEM((SLABS_PER_W * Q_LEN, HEAD_DIM), jnp.float32),
            pltpu.VMEM((SLABS_PER_W * Q_LEN,), jnp.int32),
            pltpu.VMEM((Q_LEN,), jnp.int32),
            pltpu.SemaphoreType.DMA,
            pltpu.SemaphoreType.DMA,
            pltpu.SemaphoreType.DMA,
        ],
    )
    k_out, v_out = run(k_cache.reshape(ROWS, HEAD_DIM),
                       v_cache.reshape(ROWS, HEAD_DIM),
                       input_pos,
                       k_val.reshape(VAL_ROWS, HEAD_DIM),
                       v_val.reshape(VAL_ROWS, HEAD_DIM))
    return k_out.reshape(shape4), v_out.reshape(shape4)


# speedup vs baseline: 1.0644x; 1.0580x over previous
import jax
import jax.numpy as jnp
from jax import lax
from jax.experimental import pallas as pl
from jax.experimental.pallas import tpu as pltpu
from jax.experimental.pallas import tpu_sc as plsc

MAX_B = 8
MAX_S = 2048
N_HEADS = 16
HEAD_DIM = 128
Q_LEN = 16

ROWS = MAX_B * N_HEADS * MAX_S
SLABS = MAX_B * N_HEADS
VAL_ROWS = SLABS * Q_LEN
SHAPE4 = (MAX_B, N_HEADS, MAX_S, HEAD_DIM)


_info = plsc.get_sparse_core_info()
NC, NS, L = _info.num_cores, _info.num_subcores, _info.num_lanes
NW = NC * NS
ROWS_PER_W = ROWS // NW
SLABS_PER_W = SLABS // NW
CHUNK = 256
NCHUNK = ROWS_PER_W // CHUNK
NBUF = 3


def _copy_range(src, dst, base, bufs, sem_r, sem_w):
    nbuf = len(bufs)

    def read(c, buf):
        return pltpu.make_async_copy(src.at[pl.ds(base + c * CHUNK, CHUNK)], buf, sem_r)

    def write(c, buf):
        return pltpu.make_async_copy(buf, dst.at[pl.ds(base + c * CHUNK, CHUNK)], sem_w)

    for b in range(nbuf):
        read(b, bufs[b]).start()

    def body(g):
        for b in range(nbuf):
            c = g + b
            buf = bufs[b]
            read(c, buf).wait()
            write(c, buf).start()
        for b in range(nbuf):
            c = g + b
            buf = bufs[b]
            write(c, buf).wait()

            @pl.when(c + nbuf < NCHUNK)
            def _():
                read(c + nbuf, buf).start()

    main = (NCHUNK // nbuf) * nbuf
    pl.loop(0, main, step=nbuf)(body)
    for c in range(main, NCHUNK):
        buf = bufs[c % nbuf]
        read(c, buf).wait()
        write(c, buf).start()
    for c in range(main, NCHUNK):
        write(c, bufs[c % nbuf]).wait()


def _sc_body(cache, pos, val, out,
             buf0, buf1, buf2, valbuf, idxbuf, posbuf, sem_r, sem_w, sem_s):
    wid = lax.axis_index("s") * NC + lax.axis_index("c")
    base = wid * ROWS_PER_W

    pltpu.sync_copy(pos, posbuf)
    p16 = posbuf[...]
    for s in range(SLABS_PER_W):
        slab = wid * SLABS_PER_W + s
        idxbuf[pl.ds(s * Q_LEN, Q_LEN)] = p16 + slab * MAX_S

    _copy_range(cache, out, base, (buf0, buf1, buf2), sem_r, sem_w)
    pltpu.async_copy(val.at[pl.ds(wid * SLABS_PER_W * Q_LEN,
                                  SLABS_PER_W * Q_LEN)],
                     valbuf, sem_s).wait()
    pltpu.async_copy(valbuf, out.at[idxbuf], sem_s).wait()


def _sc_update(cache, input_pos, val):
    mesh = plsc.VectorSubcoreMesh(core_axis_name="c", subcore_axis_name="s")
    run = pl.kernel(
        _sc_body,
        out_type=jax.ShapeDtypeStruct((ROWS, HEAD_DIM), jnp.float32),
        mesh=mesh,
        scratch_types=[
            pltpu.VMEM((CHUNK, HEAD_DIM), jnp.float32),
            pltpu.VMEM((CHUNK, HEAD_DIM), jnp.float32),
            pltpu.VMEM((CHUNK, HEAD_DIM), jnp.float32),
            pltpu.VMEM((SLABS_PER_W * Q_LEN, HEAD_DIM), jnp.float32),
            pltpu.VMEM((SLABS_PER_W * Q_LEN,), jnp.int32),
            pltpu.VMEM((Q_LEN,), jnp.int32),
            pltpu.SemaphoreType.DMA,
            pltpu.SemaphoreType.DMA,
            pltpu.SemaphoreType.DMA,
        ],
    )
    return run(cache.reshape(ROWS, HEAD_DIM), input_pos,
               val.reshape(VAL_ROWS, HEAD_DIM)).reshape(SHAPE4)


BLK = 16384
BLK_SLABS = BLK // MAX_S


def _tc_body(pos_ref, cache_ref, val_ref, out_ref):
    out_ref[...] = cache_ref[...]
    for s in range(BLK_SLABS):
        for i in range(Q_LEN):
            p = pos_ref[i]
            out_ref[pl.ds(s * MAX_S + p, 1), :] = val_ref[pl.ds(s * Q_LEN + i, 1), :]


def _tc_update(cache, input_pos, val):
    cache_spec = pl.BlockSpec((BLK, HEAD_DIM), lambda i: (i, 0))
    val_spec = pl.BlockSpec((BLK_SLABS * Q_LEN, HEAD_DIM), lambda i: (i, 0))
    out = pl.pallas_call(
        _tc_body,
        grid=(ROWS // BLK,),
        in_specs=[
            pl.BlockSpec(memory_space=pltpu.SMEM),
            cache_spec,
            val_spec,
        ],
        out_specs=cache_spec,
        out_shape=jax.ShapeDtypeStruct((ROWS, HEAD_DIM), jnp.float32),
    )(input_pos, cache.reshape(ROWS, HEAD_DIM), val.reshape(VAL_ROWS, HEAD_DIM))
    return out.reshape(SHAPE4)


def kernel(k_cache, v_cache, input_pos, k_val, v_val):
    v_out = _sc_update(v_cache, input_pos, v_val)
    k_out = _tc_update(k_cache, input_pos, k_val)
    return k_out, v_out


# speedup vs baseline: 1.1279x; 1.0596x over previous
import jax
import jax.numpy as jnp
from jax import lax
from jax.experimental import pallas as pl
from jax.experimental.pallas import tpu as pltpu
from jax.experimental.pallas import tpu_sc as plsc

MAX_B = 8
MAX_S = 2048
N_HEADS = 16
HEAD_DIM = 128
Q_LEN = 16

ROWS = MAX_B * N_HEADS * MAX_S
SLABS = MAX_B * N_HEADS
VAL_ROWS = SLABS * Q_LEN
SHAPE4 = (MAX_B, N_HEADS, MAX_S, HEAD_DIM)


_info = plsc.get_sparse_core_info()
NC, NS, L = _info.num_cores, _info.num_subcores, _info.num_lanes
NW = NC * NS
SLABS_PER_W = 1
SC_ROWS = NW * SLABS_PER_W * MAX_S
ROWS_PER_W = SC_ROWS // NW
CHUNK = 256
NCHUNK = ROWS_PER_W // CHUNK
NBUF = 3


def _copy_range(src, dst, base, bufs, sem_r, sem_w):
    nbuf = len(bufs)

    def read(c, buf):
        return pltpu.make_async_copy(src.at[pl.ds(base + c * CHUNK, CHUNK)], buf, sem_r)

    def write(c, buf):
        return pltpu.make_async_copy(buf, dst.at[pl.ds(base + c * CHUNK, CHUNK)], sem_w)

    for b in range(nbuf):
        read(b, bufs[b]).start()

    def body(g):
        for b in range(nbuf):
            c = g + b
            buf = bufs[b]
            read(c, buf).wait()
            write(c, buf).start()
        for b in range(nbuf):
            c = g + b
            buf = bufs[b]
            write(c, buf).wait()

            @pl.when(c + nbuf < NCHUNK)
            def _():
                read(c + nbuf, buf).start()

    main = (NCHUNK // nbuf) * nbuf
    pl.loop(0, main, step=nbuf)(body)
    for c in range(main, NCHUNK):
        buf = bufs[c % nbuf]
        read(c, buf).wait()
        write(c, buf).start()
    for c in range(main, NCHUNK):
        write(c, bufs[c % nbuf]).wait()


def _sc_body(cache, pos, val, out,
             buf0, buf1, buf2, valbuf, idxbuf, posbuf, sem_r, sem_w, sem_s):
    wid = lax.axis_index("s") * NC + lax.axis_index("c")
    base = wid * ROWS_PER_W

    pltpu.sync_copy(pos, posbuf)
    p16 = posbuf[...]
    for s in range(SLABS_PER_W):
        slab = wid * SLABS_PER_W + s
        idxbuf[pl.ds(s * Q_LEN, Q_LEN)] = p16 + slab * MAX_S

    _copy_range(cache, out, base, (buf0, buf1, buf2), sem_r, sem_w)
    pltpu.async_copy(val.at[pl.ds(wid * SLABS_PER_W * Q_LEN,
                                  SLABS_PER_W * Q_LEN)],
                     valbuf, sem_s).wait()
    pltpu.async_copy(valbuf, out.at[idxbuf], sem_s).wait()


def _sc_update_head(cache, input_pos, val):
    mesh = plsc.VectorSubcoreMesh(core_axis_name="c", subcore_axis_name="s")
    run = pl.kernel(
        _sc_body,
        out_type=jax.ShapeDtypeStruct((ROWS, HEAD_DIM), jnp.float32),
        mesh=mesh,
        scratch_types=[
            pltpu.VMEM((CHUNK, HEAD_DIM), jnp.float32),
            pltpu.VMEM((CHUNK, HEAD_DIM), jnp.float32),
            pltpu.VMEM((CHUNK, HEAD_DIM), jnp.float32),
            pltpu.VMEM((SLABS_PER_W * Q_LEN, HEAD_DIM), jnp.float32),
            pltpu.VMEM((SLABS_PER_W * Q_LEN,), jnp.int32),
            pltpu.VMEM((Q_LEN,), jnp.int32),
            pltpu.SemaphoreType.DMA,
            pltpu.SemaphoreType.DMA,
            pltpu.SemaphoreType.DMA,
        ],
    )
    return run(cache.reshape(ROWS, HEAD_DIM), input_pos,
               val.reshape(VAL_ROWS, HEAD_DIM))


BLK = 16384
BLK_SLABS = BLK // MAX_S
SC_BLKS = SC_ROWS // BLK


def _tc_body(pos_ref, cache_ref, val_ref, out_ref):
    out_ref[...] = cache_ref[...]
    for s in range(BLK_SLABS):
        for i in range(Q_LEN):
            p = pos_ref[i]
            out_ref[pl.ds(s * MAX_S + p, 1), :] = val_ref[pl.ds(s * Q_LEN + i, 1), :]


def _tc_update(cache, input_pos, val):
    cache_spec = pl.BlockSpec((BLK, HEAD_DIM), lambda i: (i, 0))
    val_spec = pl.BlockSpec((BLK_SLABS * Q_LEN, HEAD_DIM), lambda i: (i, 0))
    out = pl.pallas_call(
        _tc_body,
        grid=(ROWS // BLK,),
        in_specs=[
            pl.BlockSpec(memory_space=pltpu.SMEM),
            cache_spec,
            val_spec,
        ],
        out_specs=cache_spec,
        out_shape=jax.ShapeDtypeStruct((ROWS, HEAD_DIM), jnp.float32),
    )(input_pos, cache.reshape(ROWS, HEAD_DIM), val.reshape(VAL_ROWS, HEAD_DIM))
    return out.reshape(SHAPE4)


def _tc_body_tail(pos_ref, done_ref, cache_ref, val_ref, out_ref):
    del done_ref
    _tc_body(pos_ref, cache_ref, val_ref, out_ref)


def _tc_update_tail(done, cache, input_pos, val):
    cache_spec = pl.BlockSpec((BLK, HEAD_DIM), lambda i: (i + SC_BLKS, 0))
    val_spec = pl.BlockSpec((BLK_SLABS * Q_LEN, HEAD_DIM), lambda i: (i + SC_BLKS, 0))
    out = pl.pallas_call(
        _tc_body_tail,
        grid=((ROWS - SC_ROWS) // BLK,),
        in_specs=[
            pl.BlockSpec(memory_space=pltpu.SMEM),
            pl.BlockSpec(memory_space=pl.ANY),
            cache_spec,
            val_spec,
        ],
        out_specs=cache_spec,
        out_shape=jax.ShapeDtypeStruct((ROWS, HEAD_DIM), jnp.float32),
        input_output_aliases={1: 0},
    )(input_pos, done, cache.reshape(ROWS, HEAD_DIM), val.reshape(VAL_ROWS, HEAD_DIM))
    return out.reshape(SHAPE4)


def kernel(k_cache, v_cache, input_pos, k_val, v_val):
    v_head = _sc_update_head(v_cache, input_pos, v_val)
    k_out = _tc_update(k_cache, input_pos, k_val)
    v_out = _tc_update_tail(v_head, v_cache, input_pos, v_val)
    return k_out, v_out


# speedup vs baseline: 1.1326x; 1.0042x over previous
import jax
import jax.numpy as jnp
from jax import lax
from jax.experimental import pallas as pl
from jax.experimental.pallas import tpu as pltpu
from jax.experimental.pallas import tpu_sc as plsc

MAX_B = 8
MAX_S = 2048
N_HEADS = 16
HEAD_DIM = 128
Q_LEN = 16

ROWS = MAX_B * N_HEADS * MAX_S
SLABS = MAX_B * N_HEADS
VAL_ROWS = SLABS * Q_LEN
SHAPE4 = (MAX_B, N_HEADS, MAX_S, HEAD_DIM)


_info = plsc.get_sparse_core_info()
NC, NS, L = _info.num_cores, _info.num_subcores, _info.num_lanes
NW = NC * NS
SLABS_PER_W = 1
ACTIVE_W = 16
SC_ROWS = ACTIVE_W * SLABS_PER_W * MAX_S
ROWS_PER_W = SC_ROWS // ACTIVE_W
CHUNK = 256
NCHUNK = ROWS_PER_W // CHUNK
NBUF = 3


def _copy_range(src, dst, base, bufs, sem_r, sem_w):
    nbuf = len(bufs)

    def read(c, buf):
        return pltpu.make_async_copy(src.at[pl.ds(base + c * CHUNK, CHUNK)], buf, sem_r)

    def write(c, buf):
        return pltpu.make_async_copy(buf, dst.at[pl.ds(base + c * CHUNK, CHUNK)], sem_w)

    for b in range(nbuf):
        read(b, bufs[b]).start()

    def body(g):
        for b in range(nbuf):
            c = g + b
            buf = bufs[b]
            read(c, buf).wait()
            write(c, buf).start()
        for b in range(nbuf):
            c = g + b
            buf = bufs[b]
            write(c, buf).wait()

            @pl.when(c + nbuf < NCHUNK)
            def _():
                read(c + nbuf, buf).start()

    main = (NCHUNK // nbuf) * nbuf
    pl.loop(0, main, step=nbuf)(body)
    for c in range(main, NCHUNK):
        buf = bufs[c % nbuf]
        read(c, buf).wait()
        write(c, buf).start()
    for c in range(main, NCHUNK):
        write(c, bufs[c % nbuf]).wait()


def _sc_body(cache, pos, val, out,
             buf0, buf1, buf2, valbuf, idxbuf, posbuf, sem_r, sem_w, sem_s):
    wid = lax.axis_index("s") * NC + lax.axis_index("c")

    @pl.when(wid < ACTIVE_W)
    def _():
        base = wid * ROWS_PER_W

        pltpu.sync_copy(pos, posbuf)
        p16 = posbuf[...]
        for s in range(SLABS_PER_W):
            slab = wid * SLABS_PER_W + s
            idxbuf[pl.ds(s * Q_LEN, Q_LEN)] = p16 + slab * MAX_S

        _copy_range(cache, out, base, (buf0, buf1, buf2), sem_r, sem_w)
        pltpu.async_copy(val.at[pl.ds(wid * SLABS_PER_W * Q_LEN,
                                      SLABS_PER_W * Q_LEN)],
                         valbuf, sem_s).wait()
        pltpu.async_copy(valbuf, out.at[idxbuf], sem_s).wait()


def _sc_update_head(cache, input_pos, val):
    mesh = plsc.VectorSubcoreMesh(core_axis_name="c", subcore_axis_name="s")
    run = pl.kernel(
        _sc_body,
        out_type=jax.ShapeDtypeStruct((ROWS, HEAD_DIM), jnp.float32),
        mesh=mesh,
        scratch_types=[
            pltpu.VMEM((CHUNK, HEAD_DIM), jnp.float32),
            pltpu.VMEM((CHUNK, HEAD_DIM), jnp.float32),
            pltpu.VMEM((CHUNK, HEAD_DIM), jnp.float32),
            pltpu.VMEM((SLABS_PER_W * Q_LEN, HEAD_DIM), jnp.float32),
            pltpu.VMEM((SLABS_PER_W * Q_LEN,), jnp.int32),
            pltpu.VMEM((Q_LEN,), jnp.int32),
            pltpu.SemaphoreType.DMA,
            pltpu.SemaphoreType.DMA,
            pltpu.SemaphoreType.DMA,
        ],
    )
    return run(cache.reshape(ROWS, HEAD_DIM), input_pos,
               val.reshape(VAL_ROWS, HEAD_DIM))


BLK = 16384
BLK_SLABS = BLK // MAX_S
SC_BLKS = SC_ROWS // BLK


def _tc_body(pos_ref, cache_ref, val_ref, out_ref):
    out_ref[...] = cache_ref[...]
    for s in range(BLK_SLABS):
        for i in range(Q_LEN):
            p = pos_ref[i]
            out_ref[pl.ds(s * MAX_S + p, 1), :] = val_ref[pl.ds(s * Q_LEN + i, 1), :]


def _tc_update(cache, input_pos, val):
    cache_spec = pl.BlockSpec((BLK, HEAD_DIM), lambda i: (i, 0))
    val_spec = pl.BlockSpec((BLK_SLABS * Q_LEN, HEAD_DIM), lambda i: (i, 0))
    out = pl.pallas_call(
        _tc_body,
        grid=(ROWS // BLK,),
        in_specs=[
            pl.BlockSpec(memory_space=pltpu.SMEM),
            cache_spec,
            val_spec,
        ],
        out_specs=cache_spec,
        out_shape=jax.ShapeDtypeStruct((ROWS, HEAD_DIM), jnp.float32),
    )(input_pos, cache.reshape(ROWS, HEAD_DIM), val.reshape(VAL_ROWS, HEAD_DIM))
    return out.reshape(SHAPE4)


def _tc_body_tail(pos_ref, done_ref, cache_ref, val_ref, out_ref):
    del done_ref
    _tc_body(pos_ref, cache_ref, val_ref, out_ref)


def _tc_update_tail(done, cache, input_pos, val):
    cache_spec = pl.BlockSpec((BLK, HEAD_DIM), lambda i: (i + SC_BLKS, 0))
    val_spec = pl.BlockSpec((BLK_SLABS * Q_LEN, HEAD_DIM), lambda i: (i + SC_BLKS, 0))
    out = pl.pallas_call(
        _tc_body_tail,
        grid=((ROWS - SC_ROWS) // BLK,),
        in_specs=[
            pl.BlockSpec(memory_space=pltpu.SMEM),
            pl.BlockSpec(memory_space=pl.ANY),
            cache_spec,
            val_spec,
        ],
        out_specs=cache_spec,
        out_shape=jax.ShapeDtypeStruct((ROWS, HEAD_DIM), jnp.float32),
        input_output_aliases={1: 0},
    )(input_pos, done, cache.reshape(ROWS, HEAD_DIM), val.reshape(VAL_ROWS, HEAD_DIM))
    return out.reshape(SHAPE4)


def kernel(k_cache, v_cache, input_pos, k_val, v_val):
    v_head = _sc_update_head(v_cache, input_pos, v_val)
    k_out = _tc_update(k_cache, input_pos, k_val)
    v_out = _tc_update_tail(v_head, v_cache, input_pos, v_val)
    return k_out, v_out
